# baseline (speedup 1.0000x reference)
import functools

import jax
import jax.numpy as jnp
from jax import lax
from jax.experimental import pallas as pl
from jax.experimental.pallas import tpu as pltpu
from jax.experimental.pallas import tpu_sc as plsc

N = 8192
D = 256
K = 1024
N_ITERS = 5
DP = D + 16
BLK = 512
NBLK = N // BLK

NUM_SC = 2
NUM_SUBCORES = 16
NUM_TILES = NUM_SC * NUM_SUBCORES
ROWS_PER_TILE = N // NUM_TILES
KROWS_PER_SUBCORE = K // NUM_SUBCORES
IDX_CHUNK = 128
NCHUNK = ROWS_PER_TILE // IDX_CHUNK


def _assign_body(x_ref, c_ref, lab_ref, c2_ref, iota_ref):
    i = pl.program_id(0)
    c = c_ref[...]

    @pl.when(i == 0)
    def _():
        c2_ref[...] = lax.dot_general(
            jnp.ones((1, D), jnp.float32), c * c, (((1,), (1,)), ((), ())),
            preferred_element_type=jnp.float32)
        iota_ref[...] = lax.broadcasted_iota(
            jnp.int32, (BLK, K), 1).astype(jnp.float32)

    m = lax.dot_general(
        x_ref[...], c, (((1,), (1,)), ((), ())),
        preferred_element_type=jnp.float32)
    score = c2_ref[...] - 2.0 * m
    mn = jnp.min(score, axis=1, keepdims=True)
    cand = jnp.where(score == mn, iota_ref[...], jnp.float32(K))
    labf = jnp.min(cand, axis=1, keepdims=True)
    lab_ref[0, :, :] = labf.astype(jnp.int32)


_assign_call = pl.pallas_call(
    _assign_body,
    grid=(NBLK,),
    in_specs=[
        pl.BlockSpec((BLK, D), lambda i: (i, 0)),
        pl.BlockSpec((K, D), lambda i: (0, 0)),
    ],
    out_specs=pl.BlockSpec((1, BLK, 1), lambda i: (i, 0, 0)),
    out_shape=jax.ShapeDtypeStruct((NBLK, BLK, 1), jnp.int32),
    scratch_shapes=[
        pltpu.VMEM((1, K), jnp.float32),
        pltpu.VMEM((BLK, K), jnp.float32),
    ],
)


def _assign(x, c):
    return _assign_call(x, c).reshape(N)


def _sc_segsum_body(xa_hbm, lab_hbm, zeros_hbm, out_hbm, idx_a, idx_b, rows_v,
                    shared):
    cid = lax.axis_index("c")
    sid = lax.axis_index("s")
    wid = cid * NUM_SUBCORES + sid
    base = wid * ROWS_PER_TILE
    pltpu.sync_copy(
        zeros_hbm.at[pl.ds(sid * KROWS_PER_SUBCORE, KROWS_PER_SUBCORE)],
        shared.at[pl.ds(sid * KROWS_PER_SUBCORE, KROWS_PER_SUBCORE)])
    pltpu.sync_copy(xa_hbm.at[pl.ds(base, ROWS_PER_TILE)], rows_v)
    pltpu.sync_copy(lab_hbm.at[wid * NCHUNK], idx_a)
    pltpu.sync_copy(lab_hbm.at[wid * NCHUNK + 1], idx_b)
    plsc.subcore_barrier()
    for j, idx in enumerate((idx_a, idx_b)):
        pltpu.sync_copy(
            rows_v.at[pl.ds(j * IDX_CHUNK, IDX_CHUNK)],
            shared.at[idx],
            add=True)
    plsc.subcore_barrier()
    pltpu.sync_copy(
        shared.at[pl.ds(sid * KROWS_PER_SUBCORE, KROWS_PER_SUBCORE)],
        out_hbm.at[cid].at[pl.ds(sid * KROWS_PER_SUBCORE, KROWS_PER_SUBCORE)])


@functools.cache
def _sc_segsum():
    mesh = plsc.VectorSubcoreMesh(core_axis_name="c", subcore_axis_name="s")
    return pl.kernel(
        _sc_segsum_body,
        mesh=mesh,
        compiler_params=pltpu.CompilerParams(use_tc_tiling_on_sc=False),
        out_type=jax.ShapeDtypeStruct((NUM_SC, K, DP), jnp.float32),
        scratch_types=[
            pltpu.VMEM((IDX_CHUNK,), jnp.int32),
            pltpu.VMEM((IDX_CHUNK,), jnp.int32),
            pltpu.VMEM((ROWS_PER_TILE, DP), jnp.float32),
            pltpu.VMEM_SHARED((K, DP), jnp.float32),
        ],
    )


@jax.jit
def kernel(x):
    x = x.reshape(x.shape[0], -1)
    pad = jnp.concatenate(
        [jnp.ones((N, 1), jnp.float32),
         jnp.zeros((N, DP - D - 1), jnp.float32)], axis=1)
    xa = jnp.concatenate([x, pad], axis=1)
    zeros = jnp.zeros((K, DP), jnp.float32)
    c = x[:K]
    for _ in range(N_ITERS):
        labels = _assign(x, c)
        acc = _sc_segsum()(xa, labels.reshape(N // IDX_CHUNK, IDX_CHUNK), zeros)
        acc = acc[0] + acc[1]
        sums = acc[:, :D]
        counts = acc[:, D]
        newc = sums / jnp.maximum(counts, 1.0)[:, None]
        c = jnp.where(counts[:, None] > 0, newc, c)
    return _assign(x, c)


# speedup vs baseline: 1.1270x; 1.1270x over previous
import functools

import jax
import jax.numpy as jnp
from jax import lax
from jax.experimental import pallas as pl
from jax.experimental.pallas import tpu as pltpu
from jax.experimental.pallas import tpu_sc as plsc

N = 8192
D = 256
K = 1024
N_ITERS = 5
BLK = 512
NBLK = N // BLK

NUM_SC = 2
NUM_SUBCORES = 16
NUM_TILES = NUM_SC * NUM_SUBCORES
ROWS_PER_TILE = N // NUM_TILES
KROWS_PER_SUBCORE = K // NUM_SUBCORES
IDX_CHUNK = 128
NCHUNK = ROWS_PER_TILE // IDX_CHUNK


def _assign_body(x_ref, ct_ref, lab_ref, cnt_ref, c2_ref, iota_ref):
    i = pl.program_id(0)

    @pl.when(i == 0)
    def _():
        ct = ct_ref[...]
        c2_ref[...] = jnp.sum(ct * ct, axis=0, keepdims=True)
        iota_ref[...] = lax.broadcasted_iota(
            jnp.int32, (BLK, K), 1).astype(jnp.float32)
        cnt_ref[0, :, :] = jnp.zeros((1, K), jnp.float32)

    m = lax.dot_general(
        x_ref[...], ct_ref[...], (((1,), (0,)), ((), ())),
        preferred_element_type=jnp.float32)
    score = c2_ref[...] - 2.0 * m
    mn = jnp.min(score, axis=1, keepdims=True)
    eq = score == mn
    cand = jnp.where(eq, iota_ref[...], jnp.float32(K))
    labf = jnp.min(cand, axis=1, keepdims=True)
    lab_ref[0, :, :] = labf.astype(jnp.int32)
    onehot = labf == iota_ref[...]
    cnt_ref[0, :, :] += jnp.sum(onehot.astype(jnp.float32), axis=0,
                                keepdims=True)


_assign_call = pl.pallas_call(
    _assign_body,
    grid=(NBLK,),
    in_specs=[
        pl.BlockSpec((BLK, D), lambda i: (i, 0)),
        pl.BlockSpec((D, K), lambda i: (0, 0)),
    ],
    out_specs=[
        pl.BlockSpec((1, BLK, 1), lambda i: (i, 0, 0)),
        pl.BlockSpec((1, 1, K), lambda i: (0, 0, 0)),
    ],
    out_shape=[
        jax.ShapeDtypeStruct((NBLK, BLK, 1), jnp.int32),
        jax.ShapeDtypeStruct((1, 1, K), jnp.float32),
    ],
    scratch_shapes=[
        pltpu.VMEM((1, K), jnp.float32),
        pltpu.VMEM((BLK, K), jnp.float32),
    ],
)


def _assign(x, c):
    lab, cnt = _assign_call(x, c.T)
    return lab.reshape(N), cnt.reshape(K)


def _sc_segsum_body(x_hbm, lab_hbm, zeros_hbm, out_hbm, idx_a, idx_b, rows_v,
                    shared):
    cid = lax.axis_index("c")
    sid = lax.axis_index("s")
    wid = cid * NUM_SUBCORES + sid
    base = wid * ROWS_PER_TILE
    pltpu.sync_copy(
        zeros_hbm.at[pl.ds(sid * KROWS_PER_SUBCORE, KROWS_PER_SUBCORE)],
        shared.at[pl.ds(sid * KROWS_PER_SUBCORE, KROWS_PER_SUBCORE)])
    pltpu.sync_copy(x_hbm.at[pl.ds(base, ROWS_PER_TILE)], rows_v)
    pltpu.sync_copy(lab_hbm.at[wid * NCHUNK], idx_a)
    pltpu.sync_copy(lab_hbm.at[wid * NCHUNK + 1], idx_b)
    plsc.subcore_barrier()
    for j, idx in enumerate((idx_a, idx_b)):
        pltpu.sync_copy(
            rows_v.at[pl.ds(j * IDX_CHUNK, IDX_CHUNK)],
            shared.at[idx],
            add=True)
    plsc.subcore_barrier()
    pltpu.sync_copy(
        shared.at[pl.ds(sid * KROWS_PER_SUBCORE, KROWS_PER_SUBCORE)],
        out_hbm.at[cid].at[pl.ds(sid * KROWS_PER_SUBCORE, KROWS_PER_SUBCORE)])


@functools.cache
def _sc_segsum():
    mesh = plsc.VectorSubcoreMesh(core_axis_name="c", subcore_axis_name="s")
    return pl.kernel(
        _sc_segsum_body,
        mesh=mesh,
        compiler_params=pltpu.CompilerParams(use_tc_tiling_on_sc=False),
        out_type=jax.ShapeDtypeStruct((NUM_SC, K, D), jnp.float32),
        scratch_types=[
            pltpu.VMEM((IDX_CHUNK,), jnp.int32),
            pltpu.VMEM((IDX_CHUNK,), jnp.int32),
            pltpu.VMEM((ROWS_PER_TILE, D), jnp.float32),
            pltpu.VMEM_SHARED((K, D), jnp.float32),
        ],
    )


@jax.jit
def kernel(x):
    x = x.reshape(x.shape[0], -1)
    zeros = jnp.zeros((K, D), jnp.float32)
    c = x[:K]
    for _ in range(N_ITERS):
        labels, counts = _assign(x, c)
        acc = _sc_segsum()(x, labels.reshape(N // IDX_CHUNK, IDX_CHUNK), zeros)
        sums = acc[0] + acc[1]
        newc = sums / jnp.maximum(counts, 1.0)[:, None]
        c = jnp.where(counts[:, None] > 0, newc, c)
    return _assign(x, c)[0]


# speedup vs baseline: 1.1430x; 1.0142x over previous
import functools

import jax
import jax.numpy as jnp
from jax import lax
from jax.experimental import pallas as pl
from jax.experimental.pallas import tpu as pltpu
from jax.experimental.pallas import tpu_sc as plsc

N = 8192
D = 256
K = 1024
N_ITERS = 5
BLK = 512
NBLK = N // BLK

NUM_SC = 2
NUM_SUBCORES = 16
NUM_TILES = NUM_SC * NUM_SUBCORES
ROWS_PER_TILE = N // NUM_TILES
KROWS_PER_SUBCORE = K // NUM_SUBCORES
IDX_CHUNK = 128
NCHUNK = ROWS_PER_TILE // IDX_CHUNK


def _assign_body(x_ref, ct_ref, lab_ref, cnt_ref, c2_ref, iota_ref):
    i = pl.program_id(0)

    @pl.when(i == 0)
    def _():
        ct = ct_ref[...]
        c2_ref[...] = jnp.sum(ct * ct, axis=0, keepdims=True)
        iota_ref[...] = lax.broadcasted_iota(
            jnp.int32, (BLK, K), 1).astype(jnp.float32)
        cnt_ref[0, :, :] = jnp.zeros((1, K), jnp.float32)

    m = lax.dot_general(
        x_ref[...], ct_ref[...], (((1,), (0,)), ((), ())),
        preferred_element_type=jnp.float32)
    score = c2_ref[...] - 2.0 * m
    mn = jnp.min(score, axis=1, keepdims=True)
    eq = score == mn
    cand = jnp.where(eq, iota_ref[...], jnp.float32(K))
    labf = jnp.min(cand, axis=1, keepdims=True)
    lab_ref[0, :, :] = labf.astype(jnp.int32)
    onehot = labf == iota_ref[...]
    cnt_ref[0, :, :] += jnp.sum(onehot.astype(jnp.float32), axis=0,
                                keepdims=True)


_assign_call = pl.pallas_call(
    _assign_body,
    grid=(NBLK,),
    in_specs=[
        pl.BlockSpec((BLK, D), lambda i: (i, 0)),
        pl.BlockSpec((D, K), lambda i: (0, 0)),
    ],
    out_specs=[
        pl.BlockSpec((1, BLK, 1), lambda i: (i, 0, 0)),
        pl.BlockSpec((1, 1, K), lambda i: (0, 0, 0)),
    ],
    out_shape=[
        jax.ShapeDtypeStruct((NBLK, BLK, 1), jnp.int32),
        jax.ShapeDtypeStruct((1, 1, K), jnp.float32),
    ],
    scratch_shapes=[
        pltpu.VMEM((1, K), jnp.float32),
        pltpu.VMEM((BLK, K), jnp.float32),
    ],
)


def _assign(x, ct):
    lab, cnt = _assign_call(x, ct)
    return lab.reshape(N), cnt.reshape(K)


def _update_assign_body(x_ref, acc_ref, cntp_ref, ctp_ref,
                        lab_ref, cnt_ref, ctn_ref, c2_ref, iota_ref):
    i = pl.program_id(0)

    @pl.when(i == 0)
    def _():
        sums_t = jnp.transpose(acc_ref[0] + acc_ref[1])
        cnt = cntp_ref[0]
        newct = sums_t / jnp.maximum(cnt, 1.0)
        ctn_ref[...] = jnp.where(cnt > 0, newct, ctp_ref[...])
        ct = ctn_ref[...]
        c2_ref[...] = jnp.sum(ct * ct, axis=0, keepdims=True)
        iota_ref[...] = lax.broadcasted_iota(
            jnp.int32, (BLK, K), 1).astype(jnp.float32)
        cnt_ref[0, :, :] = jnp.zeros((1, K), jnp.float32)

    m = lax.dot_general(
        x_ref[...], ctn_ref[...], (((1,), (0,)), ((), ())),
        preferred_element_type=jnp.float32)
    score = c2_ref[...] - 2.0 * m
    mn = jnp.min(score, axis=1, keepdims=True)
    cand = jnp.where(score == mn, iota_ref[...], jnp.float32(K))
    labf = jnp.min(cand, axis=1, keepdims=True)
    lab_ref[0, :, :] = labf.astype(jnp.int32)
    onehot = labf == iota_ref[...]
    cnt_ref[0, :, :] += jnp.sum(onehot.astype(jnp.float32), axis=0,
                                keepdims=True)


_update_assign_call = pl.pallas_call(
    _update_assign_body,
    grid=(NBLK,),
    in_specs=[
        pl.BlockSpec((BLK, D), lambda i: (i, 0)),
        pl.BlockSpec((NUM_SC, K, D), lambda i: (0, 0, 0)),
        pl.BlockSpec((1, 1, K), lambda i: (0, 0, 0)),
        pl.BlockSpec((D, K), lambda i: (0, 0)),
    ],
    out_specs=[
        pl.BlockSpec((1, BLK, 1), lambda i: (i, 0, 0)),
        pl.BlockSpec((1, 1, K), lambda i: (0, 0, 0)),
        pl.BlockSpec((D, K), lambda i: (0, 0)),
    ],
    out_shape=[
        jax.ShapeDtypeStruct((NBLK, BLK, 1), jnp.int32),
        jax.ShapeDtypeStruct((1, 1, K), jnp.float32),
        jax.ShapeDtypeStruct((D, K), jnp.float32),
    ],
    scratch_shapes=[
        pltpu.VMEM((1, K), jnp.float32),
        pltpu.VMEM((BLK, K), jnp.float32),
    ],
)


def _update_assign(x, acc, counts, ctp):
    lab, cnt, ctn = _update_assign_call(
        x, acc, counts.reshape(1, 1, K), ctp)
    return lab.reshape(N), cnt.reshape(K), ctn


def _sc_segsum_body(x_hbm, lab_hbm, zeros_hbm, out_hbm, idx_a, idx_b, rows_v,
                    shared):
    cid = lax.axis_index("c")
    sid = lax.axis_index("s")
    wid = cid * NUM_SUBCORES + sid
    base = wid * ROWS_PER_TILE
    pltpu.sync_copy(
        zeros_hbm.at[pl.ds(sid * KROWS_PER_SUBCORE, KROWS_PER_SUBCORE)],
        shared.at[pl.ds(sid * KROWS_PER_SUBCORE, KROWS_PER_SUBCORE)])
    pltpu.sync_copy(x_hbm.at[pl.ds(base, ROWS_PER_TILE)], rows_v)
    pltpu.sync_copy(lab_hbm.at[wid * NCHUNK], idx_a)
    pltpu.sync_copy(lab_hbm.at[wid * NCHUNK + 1], idx_b)
    plsc.subcore_barrier()
    for j, idx in enumerate((idx_a, idx_b)):
        pltpu.sync_copy(
            rows_v.at[pl.ds(j * IDX_CHUNK, IDX_CHUNK)],
            shared.at[idx],
            add=True)
    plsc.subcore_barrier()
    pltpu.sync_copy(
        shared.at[pl.ds(sid * KROWS_PER_SUBCORE, KROWS_PER_SUBCORE)],
        out_hbm.at[cid].at[pl.ds(sid * KROWS_PER_SUBCORE, KROWS_PER_SUBCORE)])


@functools.cache
def _sc_segsum():
    mesh = plsc.VectorSubcoreMesh(core_axis_name="c", subcore_axis_name="s")
    return pl.kernel(
        _sc_segsum_body,
        mesh=mesh,
        compiler_params=pltpu.CompilerParams(use_tc_tiling_on_sc=False),
        out_type=jax.ShapeDtypeStruct((NUM_SC, K, D), jnp.float32),
        scratch_types=[
            pltpu.VMEM((IDX_CHUNK,), jnp.int32),
            pltpu.VMEM((IDX_CHUNK,), jnp.int32),
            pltpu.VMEM((ROWS_PER_TILE, D), jnp.float32),
            pltpu.VMEM_SHARED((K, D), jnp.float32),
        ],
    )


@jax.jit
def kernel(x):
    x = x.reshape(x.shape[0], -1)
    zeros = jnp.zeros((K, D), jnp.float32)
    ct = x[:K].T
    labels, counts = _assign(x, ct)
    for _ in range(N_ITERS):
        acc = _sc_segsum()(x, labels.reshape(N // IDX_CHUNK, IDX_CHUNK), zeros)
        labels, counts, ct = _update_assign(x, acc, counts, ct)
    return labels
